# Initial kernel scaffold; baseline (speedup 1.0000x reference)
import jax
import jax.numpy as jnp
from jax.experimental import pallas as pl


def kernel(n_feat, edge_index, W0, al0, ar0, b0, W1, al1, ar1, b1):
    raise NotImplementedError("write your pallas kernel here")


# speedup vs baseline: 1.0414x; 1.0414x over previous
import jax
import jax.numpy as jnp
from jax.experimental import pallas as pl

N = 10000
E = 320000
DIN = 128
HID = 128
H = 8
DH = 16


def _expand_head_vec(a):
    A = jnp.zeros((HID, H), dtype=a.dtype)
    hh = jnp.arange(HID) // DH
    A = A.at[jnp.arange(HID), hh].set(a.reshape(-1))
    return A


def _tc_pre(h, W, Al, Ar):
    def body(h_ref, w_ref, al_ref, ar_ref, feat_ref, el_ref, er_ref):
        feat = jnp.dot(h_ref[...], w_ref[...], preferred_element_type=jnp.float32)
        feat_ref[...] = feat
        el_ref[...] = jnp.dot(feat, al_ref[...], preferred_element_type=jnp.float32)
        er_ref[...] = jnp.dot(feat, ar_ref[...], preferred_element_type=jnp.float32)

    return pl.pallas_call(
        body,
        out_shape=[
            jax.ShapeDtypeStruct((N, HID), jnp.float32),
            jax.ShapeDtypeStruct((N, H), jnp.float32),
            jax.ShapeDtypeStruct((N, H), jnp.float32),
        ],
    )(h, W, Al, Ar)


def _layer(h, src, dst, W, al, ar, b, activate):
    Al = _expand_head_vec(al)
    Ar = _expand_head_vec(ar)
    feat, el, er = _tc_pre(h, W, Al, Ar)
    e = jax.nn.leaky_relu(el[src] + er[dst], negative_slope=0.2)
    ee = jnp.exp(e)
    denom = jax.ops.segment_sum(ee, dst, num_segments=N)
    alpha = ee / denom[dst]
    msg = feat.reshape(N, H, DH)[src] * alpha[:, :, None]
    rst = jax.ops.segment_sum(msg, dst, num_segments=N)
    rst = rst + b.reshape(1, H, DH)
    if activate:
        rst = jax.nn.leaky_relu(rst, negative_slope=0.01)
    return rst.reshape(N, HID)


def kernel(n_feat, edge_index, W0, al0, ar0, b0, W1, al1, ar1, b1):
    src = edge_index[0]
    dst = edge_index[1]
    h = _layer(n_feat, src, dst, W0, al0, ar0, b0, True)
    h = _layer(h, src, dst, W1, al1, ar1, b1, False)
    return h


# speedup vs baseline: 43.4645x; 41.7378x over previous
import functools

import jax
import jax.numpy as jnp
from jax import lax
from jax.experimental import pallas as pl
from jax.experimental.pallas import tpu as pltpu
from jax.experimental.pallas import tpu_sc as plsc

N = 10000
E = 320000
DIN = 128
HID = 128
H = 8
DH = 16

NC = 2
NS = 16
NW = NC * NS
EPAD = 327680
EW = EPAD // NW
NACC = 10112
ROWS_PER_TILE = NACC // NS

BB = 64
NBB = EW // BB


def _pack_attn(al, ar):
    A = jnp.zeros((HID, 16), dtype=jnp.float32)
    rows = jnp.arange(HID)
    hh = rows // DH
    A = A.at[rows, hh].set(al.reshape(-1))
    A = A.at[rows, hh + H].set(ar.reshape(-1))
    return A


def _head_splat_mat():
    cols = jnp.arange(HID) // DH
    return (jnp.arange(16)[:, None] == cols[None, :]).astype(jnp.float32)


def _tc_pre(h, W, M):
    def body(h_ref, w_ref, m_ref, feat_ref, t_ref):
        feat = jnp.dot(h_ref[...], w_ref[...], preferred_element_type=jnp.float32)
        feat_ref[...] = feat
        t_ref[...] = jnp.concatenate(
            [jnp.dot(feat, m_ref[...], preferred_element_type=jnp.float32),
             jnp.zeros((NACC - N, 16), jnp.float32)], axis=0)

    return pl.pallas_call(
        body,
        out_shape=[
            jax.ShapeDtypeStruct((N, HID), jnp.float32),
            jax.ShapeDtypeStruct((NACC, 16), jnp.float32),
        ],
    )(h, W, M)


def _sc_layer(src2d, dst2d, T, feat, z16, z128):
    mesh = plsc.VectorSubcoreMesh(core_axis_name="c", subcore_axis_name="s")

    @functools.partial(
        pl.kernel,
        out_type=[
            jax.ShapeDtypeStruct((NC, NACC, 16), jnp.float32),
            jax.ShapeDtypeStruct((NC, NACC, HID), jnp.float32),
        ],
        mesh=mesh,
        scratch_types=[
            pltpu.VMEM((1, BB), jnp.int32),
            pltpu.VMEM((1, BB), jnp.int32),
            pltpu.VMEM((BB, 16), jnp.float32),
            pltpu.VMEM((BB, 16), jnp.float32),
            pltpu.VMEM((BB, HID), jnp.float32),
            pltpu.VMEM_SHARED((NACC, 16), jnp.float32),
            pltpu.VMEM_SHARED((NACC, 16), jnp.float32),
            pltpu.VMEM_SHARED((NACC, HID), jnp.float32),
        ],
    )
    def k(src_hbm, dst_hbm, t_hbm, feat_hbm, z16_hbm, z128_hbm,
          d_hbm, r_hbm, srcv, dstv, av, bv, featv, t_sp, d_sp, r_sp):
        c = lax.axis_index("c")
        s = lax.axis_index("s")
        w = s * NC + c
        rs = pl.ds(s * ROWS_PER_TILE, ROWS_PER_TILE)
        pltpu.sync_copy(t_hbm.at[rs], t_sp.at[rs])
        pltpu.sync_copy(z16_hbm.at[rs], d_sp.at[rs])
        pltpu.sync_copy(z128_hbm.at[rs], r_sp.at[rs])
        plsc.subcore_barrier()

        shift = jnp.arange(16, dtype=jnp.int32) % H + H

        @pl.loop(0, NBB)
        def _(bi):
            blk = w * NBB + bi
            pltpu.sync_copy(src_hbm.at[pl.ds(blk, 1)], srcv)
            pltpu.sync_copy(dst_hbm.at[pl.ds(blk, 1)], dstv)
            pltpu.sync_copy(t_sp.at[srcv.at[0]], av)
            pltpu.sync_copy(t_sp.at[dstv.at[0]], bv)
            pltpu.sync_copy(feat_hbm.at[srcv.at[0]], featv)

            @pl.loop(0, BB)
            def _(i):
                er = bv[i].at[shift].get(mode="promise_in_bounds")
                e = av[i] + er
                e = jnp.where(e >= 0.0, e, 0.2 * e)
                ee = jnp.exp(e)
                av[i] = ee
                for h in range(H):
                    idx = jnp.full((16,), h, jnp.int32)
                    splat = ee.at[idx].get(mode="promise_in_bounds")
                    featv[i, pl.ds(DH * h, DH)] = (
                        featv[i, pl.ds(DH * h, DH)] * splat)

            pltpu.sync_copy(av, d_sp.at[dstv.at[0]], add=True)
            pltpu.sync_copy(featv, r_sp.at[dstv.at[0]], add=True)

        plsc.subcore_barrier()
        pltpu.sync_copy(d_sp.at[rs], d_hbm.at[c].at[rs])
        pltpu.sync_copy(r_sp.at[rs], r_hbm.at[c].at[rs])

    return k(src2d, dst2d, T, feat, z16, z128)


def _tc_post_pre(dpart, rpart, S, b, W, M):
    def body(d_ref, r_ref, s_ref, b_ref, w_ref, m_ref, feat_ref, t_ref):
        dsum = d_ref[0, :N, :] + d_ref[1, :N, :]
        dexp = jnp.dot(dsum, s_ref[...], preferred_element_type=jnp.float32)
        num = r_ref[0, :N, :] + r_ref[1, :N, :]
        rst = jnp.where(dexp != 0.0, num / dexp, 0.0) + b_ref[...]
        rst = jnp.where(rst >= 0.0, rst, 0.01 * rst)
        feat = jnp.dot(rst, w_ref[...], preferred_element_type=jnp.float32)
        feat_ref[...] = feat
        t_ref[...] = jnp.concatenate(
            [jnp.dot(feat, m_ref[...], preferred_element_type=jnp.float32),
             jnp.zeros((NACC - N, 16), jnp.float32)], axis=0)

    return pl.pallas_call(
        body,
        out_shape=[
            jax.ShapeDtypeStruct((N, HID), jnp.float32),
            jax.ShapeDtypeStruct((NACC, 16), jnp.float32),
        ],
    )(dpart, rpart, S, b.reshape(1, HID), W, M)


def _tc_post_final(dpart, rpart, S, b):
    def body(d_ref, r_ref, s_ref, b_ref, o_ref):
        dsum = d_ref[0, :N, :] + d_ref[1, :N, :]
        dexp = jnp.dot(dsum, s_ref[...], preferred_element_type=jnp.float32)
        num = r_ref[0, :N, :] + r_ref[1, :N, :]
        o_ref[...] = jnp.where(dexp != 0.0, num / dexp, 0.0) + b_ref[...]

    return pl.pallas_call(
        body, out_shape=jax.ShapeDtypeStruct((N, HID), jnp.float32),
    )(dpart, rpart, S, b.reshape(1, HID))


def kernel(n_feat, edge_index, W0, al0, ar0, b0, W1, al1, ar1, b1):
    src = edge_index[0]
    dst = edge_index[1]
    src2d = jnp.concatenate(
        [src, jnp.zeros((EPAD - E,), jnp.int32)]).reshape(EPAD // BB, BB)
    dst2d = jnp.concatenate(
        [dst, jnp.full((EPAD - E,), N, jnp.int32)]).reshape(EPAD // BB, BB)
    z16 = jnp.zeros((NACC, 16), jnp.float32)
    z128 = jnp.zeros((NACC, HID), jnp.float32)
    S = _head_splat_mat()

    feat, T0 = _tc_pre(n_feat, W0, _pack_attn(al0, ar0))
    dpart, rpart = _sc_layer(src2d, dst2d, T0, feat, z16, z128)

    feat1, T1 = _tc_post_pre(dpart, rpart, S, b0, W1, _pack_attn(al1, ar1))
    dpart1, rpart1 = _sc_layer(src2d, dst2d, T1, feat1, z16, z128)
    return _tc_post_final(dpart1, rpart1, S, b1)


# speedup vs baseline: 63.3295x; 1.4570x over previous
import functools

import jax
import jax.numpy as jnp
from jax import lax
from jax.experimental import pallas as pl
from jax.experimental.pallas import tpu as pltpu
from jax.experimental.pallas import tpu_sc as plsc

N = 10000
E = 320000
DIN = 128
HID = 128
H = 8
DH = 16

NC = 2
NS = 16
NW = NC * NS
EPAD = 327680
EW = EPAD // NW
NACC = 10112
ROWS_PER_TILE = NACC // NS

BB = 40
NBB = EW // BB


def _pack_attn(al, ar):
    A = jnp.zeros((HID, 16), dtype=jnp.float32)
    rows = jnp.arange(HID)
    hh = rows // DH
    A = A.at[rows, hh].set(al.reshape(-1))
    A = A.at[rows, hh + H].set(ar.reshape(-1))
    return A


def _head_splat_mat():
    cols = jnp.arange(HID) // DH
    return (jnp.arange(16)[:, None] == cols[None, :]).astype(jnp.float32)


def _tc_pre(h, W, M):
    def body(h_ref, w_ref, m_ref, feat_ref, t_ref):
        feat = jnp.dot(h_ref[...], w_ref[...], preferred_element_type=jnp.float32)
        feat_ref[...] = feat
        t_ref[...] = jnp.concatenate(
            [jnp.dot(feat, m_ref[...], preferred_element_type=jnp.float32),
             jnp.zeros((NACC - N, 16), jnp.float32)], axis=0)

    return pl.pallas_call(
        body,
        out_shape=[
            jax.ShapeDtypeStruct((N, HID), jnp.float32),
            jax.ShapeDtypeStruct((NACC, 16), jnp.float32),
        ],
    )(h, W, M)


def _sc_layer(src2d, dst2d, T, feat, z16, z128):
    mesh = plsc.VectorSubcoreMesh(core_axis_name="c", subcore_axis_name="s")

    @functools.partial(
        pl.kernel,
        out_type=[
            jax.ShapeDtypeStruct((NC, NACC, 16), jnp.float32),
            jax.ShapeDtypeStruct((NC, NACC, HID), jnp.float32),
        ],
        mesh=mesh,
        scratch_types=[
            pltpu.VMEM((2, BB), jnp.int32),
            pltpu.VMEM((2, BB), jnp.int32),
            pltpu.VMEM((BB, 16), jnp.float32),
            pltpu.VMEM((BB, 16), jnp.float32),
            pltpu.VMEM((2, BB, HID), jnp.float32),
            pltpu.VMEM_SHARED((NACC, 16), jnp.float32),
            pltpu.VMEM_SHARED((NACC, 16), jnp.float32),
            pltpu.VMEM_SHARED((NACC, HID), jnp.float32),
            pltpu.SemaphoreType.DMA,
            pltpu.SemaphoreType.DMA,
        ],
    )
    def k(src_hbm, dst_hbm, t_hbm, feat_hbm, z16_hbm, z128_hbm,
          d_hbm, r_hbm, srcv, dstv, av, bv, featv, t_sp, d_sp, r_sp,
          g0, g1):
        gsem = [g0, g1]
        c = lax.axis_index("c")
        s = lax.axis_index("s")
        w = s * NC + c
        rs = pl.ds(s * ROWS_PER_TILE, ROWS_PER_TILE)
        pltpu.sync_copy(t_hbm.at[rs], t_sp.at[rs])
        pltpu.sync_copy(z16_hbm.at[rs], d_sp.at[rs])
        pltpu.sync_copy(z128_hbm.at[rs], r_sp.at[rs])
        plsc.subcore_barrier()

        base = w * NBB
        shift = jnp.arange(16, dtype=jnp.int32) % H + H

        def fetch_idx(j, sl):
            blk = base + jnp.minimum(j, NBB - 1)
            pltpu.sync_copy(src_hbm.at[pl.ds(blk, 1)], srcv.at[pl.ds(sl, 1)])
            pltpu.sync_copy(dst_hbm.at[pl.ds(blk, 1)], dstv.at[pl.ds(sl, 1)])

        def issue_feat(sl):
            pltpu.async_copy(feat_hbm.at[srcv.at[sl]], featv.at[sl], gsem[sl])

        def wait_feat(sl):
            pltpu.make_async_copy(feat_hbm.at[srcv.at[sl]], featv.at[sl],
                                  gsem[sl]).wait()

        fetch_idx(0, 0)
        issue_feat(0)

        @pl.loop(0, NBB, step=2)
        def _(bi):
            for p in range(2):
                q = 1 - p
                j = bi + p
                fetch_idx(j + 1, q)
                issue_feat(q)
                pltpu.sync_copy(t_sp.at[srcv.at[p]], av)
                pltpu.sync_copy(t_sp.at[dstv.at[p]], bv)
                wait_feat(p)

                @pl.loop(0, BB)
                def _(i):
                    er = bv[i].at[shift].get(mode="promise_in_bounds")
                    e = av[i] + er
                    e = jnp.maximum(e, 0.2 * e)
                    ee = jnp.exp(e)
                    av[i] = ee
                    for h in range(H):
                        idx = jnp.full((16,), h, jnp.int32)
                        splat = ee.at[idx].get(mode="promise_in_bounds")
                        featv[p, i, pl.ds(DH * h, DH)] = (
                            featv[p, i, pl.ds(DH * h, DH)] * splat)

                pltpu.sync_copy(av, d_sp.at[dstv.at[p]], add=True)
                pltpu.sync_copy(featv.at[p], r_sp.at[dstv.at[p]], add=True)

        wait_feat(0)

        plsc.subcore_barrier()
        pltpu.sync_copy(d_sp.at[rs], d_hbm.at[c].at[rs])
        pltpu.sync_copy(r_sp.at[rs], r_hbm.at[c].at[rs])

    return k(src2d, dst2d, T, feat, z16, z128)


def _tc_post_pre(dpart, rpart, S, b, W, M):
    def body(d_ref, r_ref, s_ref, b_ref, w_ref, m_ref, feat_ref, t_ref):
        dsum = d_ref[0, :N, :] + d_ref[1, :N, :]
        dexp = jnp.dot(dsum, s_ref[...], preferred_element_type=jnp.float32)
        num = r_ref[0, :N, :] + r_ref[1, :N, :]
        rst = jnp.where(dexp != 0.0, num / dexp, 0.0) + b_ref[...]
        rst = jnp.where(rst >= 0.0, rst, 0.01 * rst)
        feat = jnp.dot(rst, w_ref[...], preferred_element_type=jnp.float32)
        feat_ref[...] = feat
        t_ref[...] = jnp.concatenate(
            [jnp.dot(feat, m_ref[...], preferred_element_type=jnp.float32),
             jnp.zeros((NACC - N, 16), jnp.float32)], axis=0)

    return pl.pallas_call(
        body,
        out_shape=[
            jax.ShapeDtypeStruct((N, HID), jnp.float32),
            jax.ShapeDtypeStruct((NACC, 16), jnp.float32),
        ],
    )(dpart, rpart, S, b.reshape(1, HID), W, M)


def _tc_post_final(dpart, rpart, S, b):
    def body(d_ref, r_ref, s_ref, b_ref, o_ref):
        dsum = d_ref[0, :N, :] + d_ref[1, :N, :]
        dexp = jnp.dot(dsum, s_ref[...], preferred_element_type=jnp.float32)
        num = r_ref[0, :N, :] + r_ref[1, :N, :]
        o_ref[...] = jnp.where(dexp != 0.0, num / dexp, 0.0) + b_ref[...]

    return pl.pallas_call(
        body, out_shape=jax.ShapeDtypeStruct((N, HID), jnp.float32),
    )(dpart, rpart, S, b.reshape(1, HID))


def kernel(n_feat, edge_index, W0, al0, ar0, b0, W1, al1, ar1, b1):
    src = edge_index[0]
    dst = edge_index[1]
    src2d = jnp.concatenate(
        [src, jnp.zeros((EPAD - E,), jnp.int32)]).reshape(EPAD // BB, BB)
    dst2d = jnp.concatenate(
        [dst, jnp.full((EPAD - E,), N, jnp.int32)]).reshape(EPAD // BB, BB)
    z16 = jnp.zeros((NACC, 16), jnp.float32)
    z128 = jnp.zeros((NACC, HID), jnp.float32)
    S = _head_splat_mat()

    feat, T0 = _tc_pre(n_feat, W0, _pack_attn(al0, ar0))
    dpart, rpart = _sc_layer(src2d, dst2d, T0, feat, z16, z128)

    feat1, T1 = _tc_post_pre(dpart, rpart, S, b0, W1, _pack_attn(al1, ar1))
    dpart1, rpart1 = _sc_layer(src2d, dst2d, T1, feat1, z16, z128)
    return _tc_post_final(dpart1, rpart1, S, b1)
